# baseline (device time: 13353 ns/iter reference)
import jax
import jax.numpy as jnp
from jax import lax
from jax.experimental import pallas as pl
from jax.experimental.pallas import tpu as pltpu

K = 4


def kernel(x):
    x = pltpu.with_memory_space_constraint(x, pltpu.MemorySpace.HBM)
    _, M, N2 = x.shape
    N = N2 // 2
    H = M // 2
    C = H // K

    def body(x_hbm, out_hbm, rows_my, loc_other, xsend, xrecv, yrecv, obuf,
             dma_sems, out_sems, xs_sems, xr_sems, ys_sems, yr_sems):
        my_x = lax.axis_index("x")
        my_y = lax.axis_index("y")
        xpeer = (1 - my_x, my_y)
        ypeer = (my_x, 1 - my_y)
        row0 = my_y * H
        other0 = (1 - my_y) * H

        barrier_sem = pltpu.get_barrier_semaphore()
        for nbr in (xpeer, ypeer):
            pl.semaphore_signal(
                barrier_sem, inc=1, device_id=nbr,
                device_id_type=pl.DeviceIdType.MESH,
            )

        dma_a = pltpu.make_async_copy(
            x_hbm.at[0, pl.ds(row0, H), :], rows_my, dma_sems.at[0]
        )
        dma_a.start()
        dma_b = pltpu.make_async_copy(
            x_hbm.at[0, pl.ds(other0, H), pl.ds(my_x * N, N)],
            loc_other,
            dma_sems.at[1],
        )
        dma_b.start()

        dma_a.wait()
        xsend[...] = rows_my[:, pl.ds((1 - my_x) * N, N)].astype(jnp.bfloat16)

        pl.semaphore_wait(barrier_sem, 2)

        xrd = []
        for k in range(K):
            r = pltpu.make_async_remote_copy(
                src_ref=xsend.at[pl.ds(k * C, C)],
                dst_ref=xrecv.at[pl.ds(k * C, C)],
                send_sem=xs_sems.at[k],
                recv_sem=xr_sems.at[k],
                device_id=xpeer,
                device_id_type=pl.DeviceIdType.MESH,
            )
            r.start()
            xrd.append(r)

        yrd = []
        for k in range(K):
            xrd[k].wait_recv()
            r = pltpu.make_async_remote_copy(
                src_ref=xrecv.at[pl.ds(k * C, C)],
                dst_ref=yrecv.at[pl.ds(k * C, C)],
                send_sem=ys_sems.at[k],
                recv_sem=yr_sems.at[k],
                device_id=ypeer,
                device_id_type=pl.DeviceIdType.MESH,
            )
            r.start()
            yrd.append(r)

        obuf[pl.ds(row0, H), :] = (
            rows_my[:, pl.ds(my_x * N, N)].astype(jnp.bfloat16) + xrecv[...]
        )
        out_my = pltpu.make_async_copy(
            obuf.at[pl.ds(row0, H)],
            out_hbm.at[pl.ds(row0, H)],
            out_sems.at[K],
        )
        out_my.start()

        dma_b.wait()
        out_other = []
        for k in range(K):
            yrd[k].wait_recv()
            obuf[pl.ds(other0 + k * C, C), :] = (
                loc_other[pl.ds(k * C, C)].astype(jnp.bfloat16)
                + yrecv[pl.ds(k * C, C)]
            )
            o = pltpu.make_async_copy(
                obuf.at[pl.ds(other0 + k * C, C)],
                out_hbm.at[pl.ds(other0 + k * C, C)],
                out_sems.at[k],
            )
            o.start()
            out_other.append(o)

        out_my.wait()
        for k in range(K):
            out_other[k].wait()
            xrd[k].wait_send()
            yrd[k].wait_send()

    return pl.pallas_call(
        body,
        out_shape=pltpu.MemorySpace.HBM((M, N), jnp.bfloat16),
        in_specs=[pl.BlockSpec(memory_space=pltpu.MemorySpace.HBM)],
        out_specs=pl.BlockSpec(memory_space=pltpu.MemorySpace.HBM),
        scratch_shapes=[
            pltpu.VMEM((H, N2), jnp.float32),
            pltpu.VMEM((H, N), jnp.float32),
            pltpu.VMEM((H, N), jnp.bfloat16),
            pltpu.VMEM((H, N), jnp.bfloat16),
            pltpu.VMEM((H, N), jnp.bfloat16),
            pltpu.VMEM((M, N), jnp.bfloat16),
            pltpu.SemaphoreType.DMA((2,)),
            pltpu.SemaphoreType.DMA((K + 1,)),
            pltpu.SemaphoreType.DMA((K,)),
            pltpu.SemaphoreType.DMA((K,)),
            pltpu.SemaphoreType.DMA((K,)),
            pltpu.SemaphoreType.DMA((K,)),
        ],
        compiler_params=pltpu.CompilerParams(collective_id=0),
    )(x)


# device time: 11290 ns/iter; 1.1827x vs baseline; 1.1827x over previous
import jax
import jax.numpy as jnp
from jax import lax
from jax.experimental import pallas as pl
from jax.experimental.pallas import tpu as pltpu

K = 4


def kernel(x):
    x = pltpu.with_memory_space_constraint(x, pltpu.MemorySpace.HBM)
    _, M, N2 = x.shape
    N = N2 // 2
    H = M // 2
    C = H // K

    def body(x_hbm, out_ref, xstage, loc_my, loc_other, xsend, xrecv, yrecv,
             s_sems, dma_sems, xs_sems, xr_sems, ys_sems, yr_sems):
        my_x = lax.axis_index("x")
        my_y = lax.axis_index("y")
        xpeer = (1 - my_x, my_y)
        ypeer = (my_x, 1 - my_y)
        row0 = my_y * H
        other0 = (1 - my_y) * H
        pcol0 = (1 - my_x) * N
        mcol0 = my_x * N

        barrier_sem = pltpu.get_barrier_semaphore()
        for nbr in (xpeer, ypeer):
            pl.semaphore_signal(
                barrier_sem, inc=1, device_id=nbr,
                device_id_type=pl.DeviceIdType.MESH,
            )

        dma_s = []
        for k in range(K):
            d = pltpu.make_async_copy(
                x_hbm.at[0, pl.ds(row0 + k * C, C), pl.ds(pcol0, N)],
                xstage.at[pl.ds(k * C, C)],
                s_sems.at[k],
            )
            d.start()
            dma_s.append(d)
        dma_m = pltpu.make_async_copy(
            x_hbm.at[0, pl.ds(row0, H), pl.ds(mcol0, N)], loc_my,
            dma_sems.at[0],
        )
        dma_m.start()
        dma_o = pltpu.make_async_copy(
            x_hbm.at[0, pl.ds(other0, H), pl.ds(mcol0, N)], loc_other,
            dma_sems.at[1],
        )
        dma_o.start()

        dma_s[0].wait()
        xsend[pl.ds(0, C)] = xstage[pl.ds(0, C)].astype(jnp.bfloat16)

        pl.semaphore_wait(barrier_sem, 2)

        xrd = []
        for k in range(K):
            r = pltpu.make_async_remote_copy(
                src_ref=xsend.at[pl.ds(k * C, C)],
                dst_ref=xrecv.at[pl.ds(k * C, C)],
                send_sem=xs_sems.at[k],
                recv_sem=xr_sems.at[k],
                device_id=xpeer,
                device_id_type=pl.DeviceIdType.MESH,
            )
            r.start()
            xrd.append(r)
            if k + 1 < K:
                dma_s[k + 1].wait()
                xsend[pl.ds((k + 1) * C, C)] = xstage[
                    pl.ds((k + 1) * C, C)
                ].astype(jnp.bfloat16)

        yrd = []
        for k in range(K):
            xrd[k].wait_recv()
            r = pltpu.make_async_remote_copy(
                src_ref=xrecv.at[pl.ds(k * C, C)],
                dst_ref=yrecv.at[pl.ds(k * C, C)],
                send_sem=ys_sems.at[k],
                recv_sem=yr_sems.at[k],
                device_id=ypeer,
                device_id_type=pl.DeviceIdType.MESH,
            )
            r.start()
            yrd.append(r)

        dma_m.wait()
        out_ref[pl.ds(row0, H), :] = (
            loc_my[...].astype(jnp.bfloat16) + xrecv[...]
        )

        dma_o.wait()
        for k in range(K):
            yrd[k].wait_recv()
            out_ref[pl.ds(other0 + k * C, C), :] = (
                loc_other[pl.ds(k * C, C)].astype(jnp.bfloat16)
                + yrecv[pl.ds(k * C, C)]
            )

        for k in range(K):
            xrd[k].wait_send()
            yrd[k].wait_send()

    return pl.pallas_call(
        body,
        out_shape=jax.ShapeDtypeStruct((M, N), jnp.bfloat16),
        in_specs=[pl.BlockSpec(memory_space=pltpu.MemorySpace.HBM)],
        out_specs=pl.BlockSpec(memory_space=pltpu.VMEM),
        scratch_shapes=[
            pltpu.VMEM((H, N), jnp.float32),
            pltpu.VMEM((H, N), jnp.float32),
            pltpu.VMEM((H, N), jnp.float32),
            pltpu.VMEM((H, N), jnp.bfloat16),
            pltpu.VMEM((H, N), jnp.bfloat16),
            pltpu.VMEM((H, N), jnp.bfloat16),
            pltpu.SemaphoreType.DMA((K,)),
            pltpu.SemaphoreType.DMA((2,)),
            pltpu.SemaphoreType.DMA((K,)),
            pltpu.SemaphoreType.DMA((K,)),
            pltpu.SemaphoreType.DMA((K,)),
            pltpu.SemaphoreType.DMA((K,)),
        ],
        compiler_params=pltpu.CompilerParams(collective_id=0),
    )(x)
